# initial kernel scaffold (unmeasured)
import jax
import jax.numpy as jnp
from jax import lax
from jax.experimental import pallas as pl
from jax.experimental.pallas import tpu as pltpu

N_DEV = 4
M_BLK = 1024
N_TOT = 8192
N_CHUNKS = 4
NC = N_TOT // N_CHUNKS


def kernel(x, w_mat):
    def body(x_ref, w_ref, out_ref, comm_ref, amax_ref,
             send_sems, recv_sems, a_send_sems, a_recv_sems):
        p = lax.axis_index("i")
        left = lax.rem(p + N_DEV - 1, N_DEV)
        right = lax.rem(p + 1, N_DEV)

        barrier_sem = pltpu.get_barrier_semaphore()
        for nbr in (left, right):
            pl.semaphore_signal(barrier_sem, inc=1, device_id=(nbr,),
                                device_id_type=pl.DeviceIdType.MESH)
        pl.semaphore_wait(barrier_sem, 2)

        def partial_chunk(j, c):
            x_blk = x_ref[pl.ds(j * M_BLK, M_BLK), :]
            return lax.dot_general(
                x_blk, w_ref[:, pl.ds(c * NC, NC)],
                (((1,), (0,)), ((), ())),
                preferred_element_type=jnp.float32)

        j0 = lax.rem(p + N_DEV - 1, N_DEV)
        for c in range(N_CHUNKS):
            comm_ref[0, :, pl.ds(c * NC, NC)] = (
                partial_chunk(j0, c).astype(jnp.bfloat16))

        amax = jnp.float32(0.0)
        for s in range(1, N_DEV):
            send_slot = (s - 1) % 2
            recv_slot = s % 2
            rdma = pltpu.make_async_remote_copy(
                src_ref=comm_ref.at[send_slot],
                dst_ref=comm_ref.at[recv_slot],
                send_sem=send_sems.at[send_slot],
                recv_sem=recv_sems.at[recv_slot],
                device_id=(right,),
                device_id_type=pl.DeviceIdType.MESH,
            )
            rdma.start()
            rdma.wait()

            j = lax.rem(p + 2 * N_DEV - 1 - s, N_DEV)
            if s < N_DEV - 1:
                for c in range(N_CHUNKS):
                    csl = pl.ds(c * NC, NC)
                    acc = (partial_chunk(j, c)
                           + comm_ref[recv_slot, :, csl].astype(jnp.float32))
                    comm_ref[recv_slot, :, csl] = acc.astype(jnp.bfloat16)
            else:
                for c in range(N_CHUNKS):
                    csl = pl.ds(c * NC, NC)
                    y = (partial_chunk(j, c)
                         + comm_ref[recv_slot, :, csl].astype(jnp.float32))
                    y = jnp.maximum(y, 0.0)
                    amax = jnp.maximum(amax, jnp.max(y))
                    out_ref[:, csl] = y

        amax_ref[pl.ds(p, 1), :] = jnp.full((1, 128), amax, jnp.float32)
        sends = []
        for off in range(1, N_DEV):
            tgt = lax.rem(p + off, N_DEV)
            a = pltpu.make_async_remote_copy(
                src_ref=amax_ref.at[pl.ds(p, 1)],
                dst_ref=amax_ref.at[pl.ds(p, 1)],
                send_sem=a_send_sems.at[off - 1],
                recv_sem=a_recv_sems.at[off - 1],
                device_id=(tgt,),
                device_id_type=pl.DeviceIdType.MESH,
            )
            a.start()
            sends.append(a)
        for off in range(1, N_DEV):
            src = lax.rem(p + N_DEV - off, N_DEV)
            r = pltpu.make_async_remote_copy(
                src_ref=amax_ref.at[pl.ds(p, 1)],
                dst_ref=amax_ref.at[pl.ds(src, 1)],
                send_sem=a_send_sems.at[off - 1],
                recv_sem=a_recv_sems.at[off - 1],
                device_id=(src,),
                device_id_type=pl.DeviceIdType.MESH,
            )
            r.wait_recv()
        for a in sends:
            a.wait_send()

        amax_g = jnp.max(amax_ref[:, :])
        scale = amax_g / 127.0
        for c in range(N_CHUNKS):
            csl = pl.ds(c * NC, NC)
            q = jnp.clip(jnp.round(out_ref[:, csl] / scale), 0.0, 127.0)
            out_ref[:, csl] = q * scale

    return pl.pallas_call(
        body,
        out_shape=jax.ShapeDtypeStruct((M_BLK, N_TOT), jnp.float32),
        in_specs=[
            pl.BlockSpec(memory_space=pltpu.VMEM),
            pl.BlockSpec(memory_space=pltpu.VMEM),
        ],
        out_specs=pl.BlockSpec(memory_space=pltpu.VMEM),
        scratch_shapes=[
            pltpu.VMEM((2, M_BLK, N_TOT), jnp.bfloat16),
            pltpu.VMEM((N_DEV, 128), jnp.float32),
            pltpu.SemaphoreType.DMA((2,)),
            pltpu.SemaphoreType.DMA((2,)),
            pltpu.SemaphoreType.DMA((N_DEV - 1,)),
            pltpu.SemaphoreType.DMA((N_DEV - 1,)),
        ],
        compiler_params=pltpu.CompilerParams(collective_id=0),
    )(x, w_mat)


# baseline (device time: 802360 ns/iter reference)
import jax
import jax.numpy as jnp
from jax import lax
from jax.experimental import pallas as pl
from jax.experimental.pallas import tpu as pltpu

N_DEV = 4
M_BLK = 1024
N_TOT = 8192
NC = 512
N_CHUNKS = N_TOT // NC


def kernel(x, w_mat):
    x = x.astype(jnp.bfloat16)
    w_mat = w_mat.astype(jnp.bfloat16)

    def body(x_ref, w_hbm, out_hbm, comm_ref, wbuf, obuf, amax_ref,
             send_sems, recv_sems, a_send_sems, a_recv_sems,
             wdma_sem, odma_sem):
        p = lax.axis_index("i")
        left = lax.rem(p + N_DEV - 1, N_DEV)
        right = lax.rem(p + 1, N_DEV)

        barrier_sem = pltpu.get_barrier_semaphore()
        for nbr in (left, right):
            pl.semaphore_signal(barrier_sem, inc=1, device_id=(nbr,),
                                device_id_type=pl.DeviceIdType.MESH)
        pl.semaphore_wait(barrier_sem, 2)

        def partial_chunk(j, c):
            cp = pltpu.make_async_copy(
                w_hbm.at[:, pl.ds(c * NC, NC)], wbuf, wdma_sem)
            cp.start()
            cp.wait()
            x_blk = x_ref[pl.ds(j * M_BLK, M_BLK), :]
            return lax.dot_general(
                x_blk, wbuf[:, :],
                (((1,), (0,)), ((), ())),
                preferred_element_type=jnp.float32)

        j0 = lax.rem(p + N_DEV - 1, N_DEV)

        def seed_chunk(c, _):
            csl = pl.ds(c * NC, NC)
            comm_ref[0, :, csl] = partial_chunk(j0, c).astype(jnp.bfloat16)
            return 0
        lax.fori_loop(0, N_CHUNKS, seed_chunk, 0)

        amax = jnp.float32(0.0)
        for s in range(1, N_DEV):
            send_slot = (s - 1) % 2
            recv_slot = s % 2
            rdma = pltpu.make_async_remote_copy(
                src_ref=comm_ref.at[send_slot],
                dst_ref=comm_ref.at[recv_slot],
                send_sem=send_sems.at[send_slot],
                recv_sem=recv_sems.at[recv_slot],
                device_id=(right,),
                device_id_type=pl.DeviceIdType.MESH,
            )
            rdma.start()
            rdma.wait()

            j = lax.rem(p + 2 * N_DEV - 1 - s, N_DEV)
            if s < N_DEV - 1:
                def upd_chunk(c, _, j=j, recv_slot=recv_slot):
                    csl = pl.ds(c * NC, NC)
                    acc = (partial_chunk(j, c)
                           + comm_ref[recv_slot, :, csl].astype(jnp.float32))
                    comm_ref[recv_slot, :, csl] = acc.astype(jnp.bfloat16)
                    return 0
                lax.fori_loop(0, N_CHUNKS, upd_chunk, 0)
            else:
                def fin_chunk(c, am, j=j, recv_slot=recv_slot):
                    csl = pl.ds(c * NC, NC)
                    y = (partial_chunk(j, c)
                         + comm_ref[recv_slot, :, csl].astype(jnp.float32))
                    y = jnp.maximum(y, 0.0)
                    comm_ref[recv_slot, :, csl] = y.astype(jnp.bfloat16)
                    return jnp.maximum(am, jnp.max(y))
                amax = lax.fori_loop(0, N_CHUNKS, fin_chunk, amax)
        final_slot = (N_DEV - 1) % 2

        amax_ref[pl.ds(p, 1), :] = jnp.full((1, 128), amax, jnp.float32)
        sends = []
        for off in range(1, N_DEV):
            tgt = lax.rem(p + off, N_DEV)
            a = pltpu.make_async_remote_copy(
                src_ref=amax_ref.at[pl.ds(p, 1)],
                dst_ref=amax_ref.at[pl.ds(p, 1)],
                send_sem=a_send_sems.at[off - 1],
                recv_sem=a_recv_sems.at[off - 1],
                device_id=(tgt,),
                device_id_type=pl.DeviceIdType.MESH,
            )
            a.start()
            sends.append(a)
        for off in range(1, N_DEV):
            src = lax.rem(p + N_DEV - off, N_DEV)
            r = pltpu.make_async_remote_copy(
                src_ref=amax_ref.at[pl.ds(p, 1)],
                dst_ref=amax_ref.at[pl.ds(src, 1)],
                send_sem=a_send_sems.at[off - 1],
                recv_sem=a_recv_sems.at[off - 1],
                device_id=(src,),
                device_id_type=pl.DeviceIdType.MESH,
            )
            r.wait_recv()
        for a in sends:
            a.wait_send()

        amax_g = jnp.max(amax_ref[:, :])
        scale = amax_g / 127.0

        def quant_chunk(c, _):
            csl = pl.ds(c * NC, NC)
            y = comm_ref[final_slot, :, csl].astype(jnp.float32)
            q = jnp.clip(jnp.round(y / scale), 0.0, 127.0)
            obuf[:, :] = q * scale
            cp = pltpu.make_async_copy(obuf, out_hbm.at[:, csl], odma_sem)
            cp.start()
            cp.wait()
            return 0
        lax.fori_loop(0, N_CHUNKS, quant_chunk, 0)

    return pl.pallas_call(
        body,
        out_shape=jax.ShapeDtypeStruct((M_BLK, N_TOT), jnp.float32),
        in_specs=[
            pl.BlockSpec(memory_space=pltpu.VMEM),
            pl.BlockSpec(memory_space=pl.ANY),
        ],
        out_specs=pl.BlockSpec(memory_space=pl.ANY),
        scratch_shapes=[
            pltpu.VMEM((2, M_BLK, N_TOT), jnp.bfloat16),
            pltpu.VMEM((1024, NC), jnp.bfloat16),
            pltpu.VMEM((M_BLK, NC), jnp.float32),
            pltpu.VMEM((N_DEV, 128), jnp.float32),
            pltpu.SemaphoreType.DMA((2,)),
            pltpu.SemaphoreType.DMA((2,)),
            pltpu.SemaphoreType.DMA((N_DEV - 1,)),
            pltpu.SemaphoreType.DMA((N_DEV - 1,)),
            pltpu.SemaphoreType.DMA,
            pltpu.SemaphoreType.DMA,
        ],
        compiler_params=pltpu.CompilerParams(
            collective_id=0,
            vmem_limit_bytes=50 * 1024 * 1024,
        ),
    )(x, w_mat)


# device time: 431123 ns/iter; 1.8611x vs baseline; 1.8611x over previous
import jax
import jax.numpy as jnp
from jax import lax
from jax.experimental import pallas as pl
from jax.experimental.pallas import tpu as pltpu

N_DEV = 4
M_BLK = 1024
N_TOT = 8192
HALF = N_TOT // 2
NC = 256
NCH = HALF // NC


def kernel(x, w_mat):
    x = x.astype(jnp.bfloat16)
    w_mat = w_mat.astype(jnp.bfloat16)

    def body(x_hbm, w_hbm, out_hbm, comm_ref, pacc_ref, xbuf, wbuf, obuf,
             amax_ref, sendR, recvR, sendL, recvL, a_send_sems, a_recv_sems,
             xsems, wsems, odma_sem):
        p = lax.axis_index("i")
        left = lax.rem(p + N_DEV - 1, N_DEV)
        right = lax.rem(p + 1, N_DEV)

        barrier_sem = pltpu.get_barrier_semaphore()
        for nbr in (left, right):
            pl.semaphore_signal(barrier_sem, inc=1, device_id=(nbr,),
                                device_id_type=pl.DeviceIdType.MESH)
        pl.semaphore_wait(barrier_sem, 2)

        def dir_partials(j, d, dst_ref):
            base = d * HALF
            xcp = pltpu.make_async_copy(
                x_hbm.at[pl.ds(j * M_BLK, M_BLK), :], xbuf.at[d], xsems.at[d])
            xcp.start()
            pltpu.make_async_copy(
                w_hbm.at[:, pl.ds(base, NC)], wbuf.at[0], wsems.at[0]
            ).start()
            xcp.wait()

            def chunk(c, _):
                buf = lax.rem(c, 2)
                nbuf = lax.rem(c + 1, 2)

                @pl.when(c + 1 < NCH)
                def _():
                    pltpu.make_async_copy(
                        w_hbm.at[:, pl.ds(base + (c + 1) * NC, NC)],
                        wbuf.at[nbuf], wsems.at[nbuf]).start()

                pltpu.make_async_copy(
                    w_hbm.at[:, pl.ds(base + c * NC, NC)],
                    wbuf.at[buf], wsems.at[buf]).wait()
                acc = lax.dot_general(
                    xbuf[d], wbuf[buf],
                    (((1,), (0,)), ((), ())),
                    preferred_element_type=jnp.float32)
                dst_ref[d, :, pl.ds(c * NC, NC)] = acc.astype(jnp.bfloat16)
                return 0

            lax.fori_loop(0, NCH, chunk, 0)

        dir_partials(left, 0, comm_ref.at[0])
        dir_partials(right, 1, comm_ref.at[0])

        amax = jnp.float32(0.0)
        for s in range(1, N_DEV):
            send_slot = (s - 1) % 2
            recv_slot = s % 2
            rdmaR = pltpu.make_async_remote_copy(
                src_ref=comm_ref.at[send_slot, 0],
                dst_ref=comm_ref.at[recv_slot, 0],
                send_sem=sendR.at[send_slot],
                recv_sem=recvR.at[recv_slot],
                device_id=(right,),
                device_id_type=pl.DeviceIdType.MESH,
            )
            rdmaL = pltpu.make_async_remote_copy(
                src_ref=comm_ref.at[send_slot, 1],
                dst_ref=comm_ref.at[recv_slot, 1],
                send_sem=sendL.at[send_slot],
                recv_sem=recvL.at[recv_slot],
                device_id=(left,),
                device_id_type=pl.DeviceIdType.MESH,
            )
            rdmaR.start()
            rdmaL.start()

            jr = lax.rem(p + 2 * N_DEV - 1 - s, N_DEV)
            jl = lax.rem(p + 1 + s, N_DEV)
            dir_partials(jr, 0, pacc_ref)
            dir_partials(jl, 1, pacc_ref)

            rdmaR.wait()
            rdmaL.wait()

            if s < N_DEV - 1:
                def add_chunk(c, _, recv_slot=recv_slot):
                    d = c // NCH
                    csl = pl.ds(lax.rem(c, NCH) * NC, NC)
                    acc = (comm_ref[recv_slot, d, :, csl].astype(jnp.float32)
                           + pacc_ref[d, :, csl].astype(jnp.float32))
                    comm_ref[recv_slot, d, :, csl] = acc.astype(jnp.bfloat16)
                    return 0
                lax.fori_loop(0, 2 * NCH, add_chunk, 0)
            else:
                def fin_chunk(c, am, recv_slot=recv_slot):
                    d = c // NCH
                    csl = pl.ds(lax.rem(c, NCH) * NC, NC)
                    y = (comm_ref[recv_slot, d, :, csl].astype(jnp.float32)
                         + pacc_ref[d, :, csl].astype(jnp.float32))
                    y = jnp.maximum(y, 0.0)
                    comm_ref[recv_slot, d, :, csl] = y.astype(jnp.bfloat16)
                    return jnp.maximum(am, jnp.max(y))
                amax = lax.fori_loop(0, 2 * NCH, fin_chunk, amax)
        final_slot = (N_DEV - 1) % 2

        amax_ref[pl.ds(p, 1), :] = jnp.full((1, 128), amax, jnp.float32)
        sends = []
        for off in range(1, N_DEV):
            tgt = lax.rem(p + off, N_DEV)
            a = pltpu.make_async_remote_copy(
                src_ref=amax_ref.at[pl.ds(p, 1)],
                dst_ref=amax_ref.at[pl.ds(p, 1)],
                send_sem=a_send_sems.at[off - 1],
                recv_sem=a_recv_sems.at[off - 1],
                device_id=(tgt,),
                device_id_type=pl.DeviceIdType.MESH,
            )
            a.start()
            sends.append(a)
        for off in range(1, N_DEV):
            src = lax.rem(p + N_DEV - off, N_DEV)
            r = pltpu.make_async_remote_copy(
                src_ref=amax_ref.at[pl.ds(p, 1)],
                dst_ref=amax_ref.at[pl.ds(src, 1)],
                send_sem=a_send_sems.at[off - 1],
                recv_sem=a_recv_sems.at[off - 1],
                device_id=(src,),
                device_id_type=pl.DeviceIdType.MESH,
            )
            r.wait_recv()
        for a in sends:
            a.wait_send()

        amax_g = jnp.max(amax_ref[:, :])
        scale = amax_g / 127.0

        def quant_chunk(c, _):
            d = c // NCH
            csl = pl.ds(lax.rem(c, NCH) * NC, NC)
            y = comm_ref[final_slot, d, :, csl].astype(jnp.float32)
            q = jnp.clip(jnp.round(y / scale), 0.0, 127.0)
            obuf[:, :] = q * scale
            cp = pltpu.make_async_copy(
                obuf,
                out_hbm.at[:, pl.ds(d * HALF + lax.rem(c, NCH) * NC, NC)],
                odma_sem)
            cp.start()
            cp.wait()
            return 0
        lax.fori_loop(0, 2 * NCH, quant_chunk, 0)

    return pl.pallas_call(
        body,
        out_shape=jax.ShapeDtypeStruct((M_BLK, N_TOT), jnp.float32),
        in_specs=[
            pl.BlockSpec(memory_space=pl.ANY),
            pl.BlockSpec(memory_space=pl.ANY),
        ],
        out_specs=pl.BlockSpec(memory_space=pl.ANY),
        scratch_shapes=[
            pltpu.VMEM((2, 2, M_BLK, HALF), jnp.bfloat16),
            pltpu.VMEM((2, M_BLK, HALF), jnp.bfloat16),
            pltpu.VMEM((2, M_BLK, 1024), jnp.bfloat16),
            pltpu.VMEM((2, 1024, NC), jnp.bfloat16),
            pltpu.VMEM((M_BLK, NC), jnp.float32),
            pltpu.VMEM((N_DEV, 128), jnp.float32),
            pltpu.SemaphoreType.DMA((2,)),
            pltpu.SemaphoreType.DMA((2,)),
            pltpu.SemaphoreType.DMA((2,)),
            pltpu.SemaphoreType.DMA((2,)),
            pltpu.SemaphoreType.DMA((N_DEV - 1,)),
            pltpu.SemaphoreType.DMA((N_DEV - 1,)),
            pltpu.SemaphoreType.DMA((2,)),
            pltpu.SemaphoreType.DMA((2,)),
            pltpu.SemaphoreType.DMA,
        ],
        compiler_params=pltpu.CompilerParams(
            collective_id=0,
            vmem_limit_bytes=63 * 1024 * 1024,
        ),
    )(x, w_mat)


# device time: 401098 ns/iter; 2.0004x vs baseline; 1.0749x over previous
import jax
import jax.numpy as jnp
from jax import lax
from jax.experimental import pallas as pl
from jax.experimental.pallas import tpu as pltpu

N_DEV = 4
M_BLK = 1024
N_TOT = 8192
HALF = N_TOT // 2
NC = 256
NCH = HALF // NC


def kernel(x, w_mat):
    x = x.astype(jnp.bfloat16)
    w_mat = w_mat.astype(jnp.bfloat16)

    def body(x_hbm, w_hbm, out_hbm, comm_ref, pacc_ref, xbuf, wbuf, obuf,
             amax_ref, sendR, recvR, sendL, recvL, a_send_sems, a_recv_sems,
             xsems, wsems, odma_sems):
        p = lax.axis_index("i")
        left = lax.rem(p + N_DEV - 1, N_DEV)
        right = lax.rem(p + 1, N_DEV)

        barrier_sem = pltpu.get_barrier_semaphore()
        for nbr in (left, right):
            pl.semaphore_signal(barrier_sem, inc=1, device_id=(nbr,),
                                device_id_type=pl.DeviceIdType.MESH)
        pl.semaphore_wait(barrier_sem, 2)

        def dir_partials(j, d, dst_ref):
            base = d * HALF
            xcp = pltpu.make_async_copy(
                x_hbm.at[pl.ds(j * M_BLK, M_BLK), :], xbuf.at[d], xsems.at[d])
            xcp.start()
            pltpu.make_async_copy(
                w_hbm.at[:, pl.ds(base, NC)], wbuf.at[0], wsems.at[0]
            ).start()
            xcp.wait()

            def chunk(c, _):
                buf = lax.rem(c, 2)
                nbuf = lax.rem(c + 1, 2)

                @pl.when(c + 1 < NCH)
                def _():
                    pltpu.make_async_copy(
                        w_hbm.at[:, pl.ds(base + (c + 1) * NC, NC)],
                        wbuf.at[nbuf], wsems.at[nbuf]).start()

                pltpu.make_async_copy(
                    w_hbm.at[:, pl.ds(base + c * NC, NC)],
                    wbuf.at[buf], wsems.at[buf]).wait()
                acc = lax.dot_general(
                    xbuf[d], wbuf[buf],
                    (((1,), (0,)), ((), ())),
                    preferred_element_type=jnp.float32)
                dst_ref[d, :, pl.ds(c * NC, NC)] = acc.astype(jnp.bfloat16)
                return 0

            lax.fori_loop(0, NCH, chunk, 0)

        dir_partials(left, 0, comm_ref.at[0])
        dir_partials(right, 1, comm_ref.at[0])

        amax = jnp.float32(0.0)
        for s in range(1, N_DEV):
            send_slot = (s - 1) % 2
            recv_slot = s % 2
            rdmaR = pltpu.make_async_remote_copy(
                src_ref=comm_ref.at[send_slot, 0],
                dst_ref=comm_ref.at[recv_slot, 0],
                send_sem=sendR.at[send_slot],
                recv_sem=recvR.at[recv_slot],
                device_id=(right,),
                device_id_type=pl.DeviceIdType.MESH,
            )
            rdmaL = pltpu.make_async_remote_copy(
                src_ref=comm_ref.at[send_slot, 1],
                dst_ref=comm_ref.at[recv_slot, 1],
                send_sem=sendL.at[send_slot],
                recv_sem=recvL.at[recv_slot],
                device_id=(left,),
                device_id_type=pl.DeviceIdType.MESH,
            )
            rdmaR.start()
            rdmaL.start()

            jr = lax.rem(p + 2 * N_DEV - 1 - s, N_DEV)
            jl = lax.rem(p + 1 + s, N_DEV)
            dir_partials(jr, 0, pacc_ref)
            dir_partials(jl, 1, pacc_ref)

            rdmaR.wait()
            rdmaL.wait()

            if s < N_DEV - 1:
                def add_chunk(c, _, recv_slot=recv_slot):
                    d = c // NCH
                    csl = pl.ds(lax.rem(c, NCH) * NC, NC)
                    acc = (comm_ref[recv_slot, d, :, csl].astype(jnp.float32)
                           + pacc_ref[d, :, csl].astype(jnp.float32))
                    comm_ref[recv_slot, d, :, csl] = acc.astype(jnp.bfloat16)
                    return 0
                lax.fori_loop(0, 2 * NCH, add_chunk, 0)
            else:
                def fin_chunk(c, am, recv_slot=recv_slot):
                    d = c // NCH
                    csl = pl.ds(lax.rem(c, NCH) * NC, NC)
                    y = (comm_ref[recv_slot, d, :, csl].astype(jnp.float32)
                         + pacc_ref[d, :, csl].astype(jnp.float32))
                    y = jnp.maximum(y, 0.0)
                    comm_ref[recv_slot, d, :, csl] = y.astype(jnp.bfloat16)
                    return jnp.maximum(am, jnp.max(y))
                amax = lax.fori_loop(0, 2 * NCH, fin_chunk, amax)
        final_slot = (N_DEV - 1) % 2

        amax_ref[pl.ds(p, 1), :] = jnp.full((1, 128), amax, jnp.float32)
        sends = []
        for off in range(1, N_DEV):
            tgt = lax.rem(p + off, N_DEV)
            a = pltpu.make_async_remote_copy(
                src_ref=amax_ref.at[pl.ds(p, 1)],
                dst_ref=amax_ref.at[pl.ds(p, 1)],
                send_sem=a_send_sems.at[off - 1],
                recv_sem=a_recv_sems.at[off - 1],
                device_id=(tgt,),
                device_id_type=pl.DeviceIdType.MESH,
            )
            a.start()
            sends.append(a)
        for off in range(1, N_DEV):
            src = lax.rem(p + N_DEV - off, N_DEV)
            r = pltpu.make_async_remote_copy(
                src_ref=amax_ref.at[pl.ds(p, 1)],
                dst_ref=amax_ref.at[pl.ds(src, 1)],
                send_sem=a_send_sems.at[off - 1],
                recv_sem=a_recv_sems.at[off - 1],
                device_id=(src,),
                device_id_type=pl.DeviceIdType.MESH,
            )
            r.wait_recv()
        for a in sends:
            a.wait_send()

        amax_g = jnp.max(amax_ref[:, :])
        scale = amax_g / 127.0

        cps = []
        for c in range(2 * NCH):
            d = c // NCH
            b = c % 2
            csl = pl.ds((c % NCH) * NC, NC)
            if c >= 2:
                cps[c - 2].wait()
            y = comm_ref[final_slot, d, :, csl].astype(jnp.float32)
            q = jnp.clip(jnp.round(y / scale), 0.0, 127.0)
            obuf[b] = (q * scale).astype(jnp.bfloat16)
            cp = pltpu.make_async_copy(
                obuf.at[b],
                out_hbm.at[:, pl.ds(d * HALF + (c % NCH) * NC, NC)],
                odma_sems.at[b])
            cp.start()
            cps.append(cp)
        cps[-2].wait()
        cps[-1].wait()

    return pl.pallas_call(
        body,
        out_shape=jax.ShapeDtypeStruct((M_BLK, N_TOT), jnp.bfloat16),
        in_specs=[
            pl.BlockSpec(memory_space=pl.ANY),
            pl.BlockSpec(memory_space=pl.ANY),
        ],
        out_specs=pl.BlockSpec(memory_space=pl.ANY),
        scratch_shapes=[
            pltpu.VMEM((2, 2, M_BLK, HALF), jnp.bfloat16),
            pltpu.VMEM((2, M_BLK, HALF), jnp.bfloat16),
            pltpu.VMEM((2, M_BLK, 1024), jnp.bfloat16),
            pltpu.VMEM((2, 1024, NC), jnp.bfloat16),
            pltpu.VMEM((2, M_BLK, NC), jnp.bfloat16),
            pltpu.VMEM((N_DEV, 128), jnp.float32),
            pltpu.SemaphoreType.DMA((2,)),
            pltpu.SemaphoreType.DMA((2,)),
            pltpu.SemaphoreType.DMA((2,)),
            pltpu.SemaphoreType.DMA((2,)),
            pltpu.SemaphoreType.DMA((N_DEV - 1,)),
            pltpu.SemaphoreType.DMA((N_DEV - 1,)),
            pltpu.SemaphoreType.DMA((2,)),
            pltpu.SemaphoreType.DMA((2,)),
            pltpu.SemaphoreType.DMA((2,)),
        ],
        compiler_params=pltpu.CompilerParams(
            collective_id=0,
            vmem_limit_bytes=63 * 1024 * 1024,
        ),
    )(x, w_mat)


# device time: 386192 ns/iter; 2.0776x vs baseline; 1.0386x over previous
import jax
import jax.numpy as jnp
from jax import lax
from jax.experimental import pallas as pl
from jax.experimental.pallas import tpu as pltpu

N_DEV = 4
M_BLK = 1024
N_TOT = 8192
HALF = N_TOT // 2
NC = 256
NCH = HALF // NC


def kernel(x, w_mat):
    x = x.astype(jnp.bfloat16)

    def body(x_hbm, w_hbm, out_hbm, comm_ref, pacc_ref, xbuf, wbuf, obuf,
             amax_ref, sendR, recvR, sendL, recvL, a_send_sems, a_recv_sems,
             xsems, wsems, odma_sems):
        p = lax.axis_index("i")
        left = lax.rem(p + N_DEV - 1, N_DEV)
        right = lax.rem(p + 1, N_DEV)

        barrier_sem = pltpu.get_barrier_semaphore()
        for nbr in (left, right):
            pl.semaphore_signal(barrier_sem, inc=1, device_id=(nbr,),
                                device_id_type=pl.DeviceIdType.MESH)
        pl.semaphore_wait(barrier_sem, 2)

        def dir_partials(j, d, dst_ref):
            base = d * HALF
            xcp = pltpu.make_async_copy(
                x_hbm.at[pl.ds(j * M_BLK, M_BLK), :], xbuf.at[d], xsems.at[d])
            xcp.start()
            pltpu.make_async_copy(
                w_hbm.at[:, pl.ds(base, NC)], wbuf.at[0], wsems.at[0]
            ).start()
            xcp.wait()

            def chunk(c, _):
                buf = lax.rem(c, 2)
                nbuf = lax.rem(c + 1, 2)

                @pl.when(c + 1 < NCH)
                def _():
                    pltpu.make_async_copy(
                        w_hbm.at[:, pl.ds(base + (c + 1) * NC, NC)],
                        wbuf.at[nbuf], wsems.at[nbuf]).start()

                pltpu.make_async_copy(
                    w_hbm.at[:, pl.ds(base + c * NC, NC)],
                    wbuf.at[buf], wsems.at[buf]).wait()
                acc = lax.dot_general(
                    xbuf[d], wbuf[buf].astype(jnp.bfloat16),
                    (((1,), (0,)), ((), ())),
                    preferred_element_type=jnp.float32)
                dst_ref[d, :, pl.ds(c * NC, NC)] = acc.astype(jnp.bfloat16)
                return 0

            lax.fori_loop(0, NCH, chunk, 0)

        dir_partials(left, 0, comm_ref.at[0])
        dir_partials(right, 1, comm_ref.at[0])

        amax = jnp.float32(0.0)
        for s in range(1, N_DEV):
            send_slot = (s - 1) % 2
            recv_slot = s % 2
            rdmaR = pltpu.make_async_remote_copy(
                src_ref=comm_ref.at[send_slot, 0],
                dst_ref=comm_ref.at[recv_slot, 0],
                send_sem=sendR.at[send_slot],
                recv_sem=recvR.at[recv_slot],
                device_id=(right,),
                device_id_type=pl.DeviceIdType.MESH,
            )
            rdmaL = pltpu.make_async_remote_copy(
                src_ref=comm_ref.at[send_slot, 1],
                dst_ref=comm_ref.at[recv_slot, 1],
                send_sem=sendL.at[send_slot],
                recv_sem=recvL.at[recv_slot],
                device_id=(left,),
                device_id_type=pl.DeviceIdType.MESH,
            )
            rdmaR.start()
            rdmaL.start()

            jr = lax.rem(p + 2 * N_DEV - 1 - s, N_DEV)
            jl = lax.rem(p + 1 + s, N_DEV)
            dir_partials(jr, 0, pacc_ref)
            dir_partials(jl, 1, pacc_ref)

            rdmaR.wait()
            rdmaL.wait()

            if s < N_DEV - 1:
                def add_chunk(c, _, recv_slot=recv_slot):
                    d = c // NCH
                    csl = pl.ds(lax.rem(c, NCH) * NC, NC)
                    acc = (comm_ref[recv_slot, d, :, csl].astype(jnp.float32)
                           + pacc_ref[d, :, csl].astype(jnp.float32))
                    comm_ref[recv_slot, d, :, csl] = acc.astype(jnp.bfloat16)
                    return 0
                lax.fori_loop(0, 2 * NCH, add_chunk, 0)
            else:
                def fin_chunk(c, am, recv_slot=recv_slot):
                    d = c // NCH
                    csl = pl.ds(lax.rem(c, NCH) * NC, NC)
                    y = (comm_ref[recv_slot, d, :, csl].astype(jnp.float32)
                         + pacc_ref[d, :, csl].astype(jnp.float32))
                    y = jnp.maximum(y, 0.0)
                    comm_ref[recv_slot, d, :, csl] = y.astype(jnp.bfloat16)
                    return jnp.maximum(am, jnp.max(y))
                amax = lax.fori_loop(0, 2 * NCH, fin_chunk, amax)
        final_slot = (N_DEV - 1) % 2

        amax_ref[pl.ds(p, 1), :] = jnp.full((1, 128), amax, jnp.float32)
        sends = []
        for off in range(1, N_DEV):
            tgt = lax.rem(p + off, N_DEV)
            a = pltpu.make_async_remote_copy(
                src_ref=amax_ref.at[pl.ds(p, 1)],
                dst_ref=amax_ref.at[pl.ds(p, 1)],
                send_sem=a_send_sems.at[off - 1],
                recv_sem=a_recv_sems.at[off - 1],
                device_id=(tgt,),
                device_id_type=pl.DeviceIdType.MESH,
            )
            a.start()
            sends.append(a)
        for off in range(1, N_DEV):
            src = lax.rem(p + N_DEV - off, N_DEV)
            r = pltpu.make_async_remote_copy(
                src_ref=amax_ref.at[pl.ds(p, 1)],
                dst_ref=amax_ref.at[pl.ds(src, 1)],
                send_sem=a_send_sems.at[off - 1],
                recv_sem=a_recv_sems.at[off - 1],
                device_id=(src,),
                device_id_type=pl.DeviceIdType.MESH,
            )
            r.wait_recv()
        for a in sends:
            a.wait_send()

        amax_g = jnp.max(amax_ref[:, :])
        scale = amax_g / 127.0

        cps = []
        for c in range(2 * NCH):
            d = c // NCH
            b = c % 2
            csl = pl.ds((c % NCH) * NC, NC)
            if c >= 2:
                cps[c - 2].wait()
            y = comm_ref[final_slot, d, :, csl].astype(jnp.float32)
            q = jnp.clip(jnp.round(y / scale), 0.0, 127.0)
            obuf[b] = (q * scale).astype(jnp.bfloat16)
            cp = pltpu.make_async_copy(
                obuf.at[b],
                out_hbm.at[:, pl.ds(d * HALF + (c % NCH) * NC, NC)],
                odma_sems.at[b])
            cp.start()
            cps.append(cp)
        cps[-2].wait()
        cps[-1].wait()

    return pl.pallas_call(
        body,
        out_shape=jax.ShapeDtypeStruct((M_BLK, N_TOT), jnp.bfloat16),
        in_specs=[
            pl.BlockSpec(memory_space=pl.ANY),
            pl.BlockSpec(memory_space=pl.ANY),
        ],
        out_specs=pl.BlockSpec(memory_space=pl.ANY),
        scratch_shapes=[
            pltpu.VMEM((2, 2, M_BLK, HALF), jnp.bfloat16),
            pltpu.VMEM((2, M_BLK, HALF), jnp.bfloat16),
            pltpu.VMEM((2, M_BLK, 1024), jnp.bfloat16),
            pltpu.VMEM((2, 1024, NC), jnp.float32),
            pltpu.VMEM((2, M_BLK, NC), jnp.bfloat16),
            pltpu.VMEM((N_DEV, 128), jnp.float32),
            pltpu.SemaphoreType.DMA((2,)),
            pltpu.SemaphoreType.DMA((2,)),
            pltpu.SemaphoreType.DMA((2,)),
            pltpu.SemaphoreType.DMA((2,)),
            pltpu.SemaphoreType.DMA((N_DEV - 1,)),
            pltpu.SemaphoreType.DMA((N_DEV - 1,)),
            pltpu.SemaphoreType.DMA((2,)),
            pltpu.SemaphoreType.DMA((2,)),
            pltpu.SemaphoreType.DMA((2,)),
        ],
        compiler_params=pltpu.CompilerParams(
            collective_id=0,
            vmem_limit_bytes=63 * 1024 * 1024,
        ),
    )(x, w_mat)


# device time: 363262 ns/iter; 2.2088x vs baseline; 1.0631x over previous
import jax
import jax.numpy as jnp
from jax import lax
from jax.experimental import pallas as pl
from jax.experimental.pallas import tpu as pltpu

N_DEV = 4
M_BLK = 1024
N_TOT = 8192
HALF = N_TOT // 2
NC = 256
NCH = HALF // NC
SEED_SUB = NCH // 2


def kernel(x, w_mat):
    x = x.astype(jnp.bfloat16)

    def body(x_hbm, w_hbm, out_hbm, comm_ref, pacc_ref, xbuf, wbuf, obuf,
             amax_ref, sendR, recvR, sendL, recvL, s1send, s1recv,
             a_send_sems, a_recv_sems, xsems, wsems, odma_sems):
        p = lax.axis_index("i")
        left = lax.rem(p + N_DEV - 1, N_DEV)
        right = lax.rem(p + 1, N_DEV)

        barrier_sem = pltpu.get_barrier_semaphore()
        for nbr in (left, right):
            pl.semaphore_signal(barrier_sem, inc=1, device_id=(nbr,),
                                device_id_type=pl.DeviceIdType.MESH)
        pl.semaphore_wait(barrier_sem, 2)

        def dir_partials(j, d, dst_ref, c0=0, c1=NCH, load_x=True):
            base = d * HALF
            if load_x:
                xcp = pltpu.make_async_copy(
                    x_hbm.at[pl.ds(j * M_BLK, M_BLK), :], xbuf.at[d],
                    xsems.at[d])
                xcp.start()
            pltpu.make_async_copy(
                w_hbm.at[:, pl.ds(base + c0 * NC, NC)],
                wbuf.at[c0 % 2], wsems.at[c0 % 2]).start()
            if load_x:
                xcp.wait()

            def chunk(c, _):
                buf = lax.rem(c, 2)
                nbuf = lax.rem(c + 1, 2)

                @pl.when(c + 1 < c1)
                def _():
                    pltpu.make_async_copy(
                        w_hbm.at[:, pl.ds(base + (c + 1) * NC, NC)],
                        wbuf.at[nbuf], wsems.at[nbuf]).start()

                pltpu.make_async_copy(
                    w_hbm.at[:, pl.ds(base + c * NC, NC)],
                    wbuf.at[buf], wsems.at[buf]).wait()
                acc = lax.dot_general(
                    xbuf[d], wbuf[buf].astype(jnp.bfloat16),
                    (((1,), (0,)), ((), ())),
                    preferred_element_type=jnp.float32)
                dst_ref[d, :, pl.ds(c * NC, NC)] = acc.astype(jnp.bfloat16)
                return 0

            lax.fori_loop(c0, c1, chunk, 0)

        def hop1_sub(d, sub, nbr):
            i = d * 2 + sub
            cols = pl.ds(sub * SEED_SUB * NC, SEED_SUB * NC)
            return pltpu.make_async_remote_copy(
                src_ref=comm_ref.at[0, d, :, cols],
                dst_ref=comm_ref.at[1, d, :, cols],
                send_sem=s1send.at[i],
                recv_sem=s1recv.at[i],
                device_id=(nbr,),
                device_id_type=pl.DeviceIdType.MESH,
            )

        def hop_full(s, d, nbr):
            send_slot = (s - 1) % 2
            recv_slot = s % 2
            ssem, rsem = (sendR, recvR) if d == 0 else (sendL, recvL)
            return pltpu.make_async_remote_copy(
                src_ref=comm_ref.at[send_slot, d],
                dst_ref=comm_ref.at[recv_slot, d],
                send_sem=ssem.at[send_slot],
                recv_sem=rsem.at[recv_slot],
                device_id=(nbr,),
                device_id_type=pl.DeviceIdType.MESH,
            )

        hop1 = {}
        dir_partials(left, 0, comm_ref.at[0], 0, SEED_SUB)
        hop1[(0, 0)] = hop1_sub(0, 0, right)
        hop1[(0, 0)].start()
        dir_partials(right, 1, comm_ref.at[0], 0, SEED_SUB)
        hop1[(1, 0)] = hop1_sub(1, 0, left)
        hop1[(1, 0)].start()
        dir_partials(left, 0, comm_ref.at[0], SEED_SUB, NCH, load_x=False)
        hop1[(0, 1)] = hop1_sub(0, 1, right)
        hop1[(0, 1)].start()
        dir_partials(right, 1, comm_ref.at[0], SEED_SUB, NCH, load_x=False)
        hop1[(1, 1)] = hop1_sub(1, 1, left)
        hop1[(1, 1)].start()

        def add_dir(d, recv_slot):
            def add_chunk(c, _):
                csl = pl.ds(c * NC, NC)
                acc = (comm_ref[recv_slot, d, :, csl].astype(jnp.float32)
                       + pacc_ref[d, :, csl].astype(jnp.float32))
                comm_ref[recv_slot, d, :, csl] = acc.astype(jnp.bfloat16)
                return 0
            lax.fori_loop(0, NCH, add_chunk, 0)

        amax = jnp.float32(0.0)
        nxt = {}
        for s in range(1, N_DEV):
            recv_slot = s % 2
            jr = lax.rem(p + 2 * N_DEV - 1 - s, N_DEV)
            jl = lax.rem(p + 1 + s, N_DEV)
            dir_partials(jr, 0, pacc_ref)
            dir_partials(jl, 1, pacc_ref)

            if s == 1:
                waitsR = [hop1[(0, 0)], hop1[(0, 1)]]
                waitsL = [hop1[(1, 0)], hop1[(1, 1)]]
            else:
                waitsR = [nxt[0]]
                waitsL = [nxt[1]]

            if s < N_DEV - 1:
                for d, waits, nbr in ((0, waitsR, right), (1, waitsL, left)):
                    for rd in waits:
                        rd.wait()
                    add_dir(d, recv_slot)
                    nxt[d] = hop_full(s + 1, d, nbr)
                    nxt[d].start()
            else:
                for rd in waitsR + waitsL:
                    rd.wait()

                def fin_chunk(c, am):
                    d = c // NCH
                    csl = pl.ds(lax.rem(c, NCH) * NC, NC)
                    y = (comm_ref[recv_slot, d, :, csl].astype(jnp.float32)
                         + pacc_ref[d, :, csl].astype(jnp.float32))
                    y = jnp.maximum(y, 0.0)
                    comm_ref[recv_slot, d, :, csl] = y.astype(jnp.bfloat16)
                    return jnp.maximum(am, jnp.max(y))
                amax = lax.fori_loop(0, 2 * NCH, fin_chunk, amax)
        final_slot = (N_DEV - 1) % 2

        amax_ref[pl.ds(p, 1), :] = jnp.full((1, 128), amax, jnp.float32)
        sends = []
        for off in range(1, N_DEV):
            tgt = lax.rem(p + off, N_DEV)
            a = pltpu.make_async_remote_copy(
                src_ref=amax_ref.at[pl.ds(p, 1)],
                dst_ref=amax_ref.at[pl.ds(p, 1)],
                send_sem=a_send_sems.at[off - 1],
                recv_sem=a_recv_sems.at[off - 1],
                device_id=(tgt,),
                device_id_type=pl.DeviceIdType.MESH,
            )
            a.start()
            sends.append(a)
        for off in range(1, N_DEV):
            src = lax.rem(p + N_DEV - off, N_DEV)
            r = pltpu.make_async_remote_copy(
                src_ref=amax_ref.at[pl.ds(p, 1)],
                dst_ref=amax_ref.at[pl.ds(src, 1)],
                send_sem=a_send_sems.at[off - 1],
                recv_sem=a_recv_sems.at[off - 1],
                device_id=(src,),
                device_id_type=pl.DeviceIdType.MESH,
            )
            r.wait_recv()
        for a in sends:
            a.wait_send()

        amax_g = jnp.max(amax_ref[:, :])
        scale = amax_g / 127.0

        cps = []
        for c in range(2 * NCH):
            d = c // NCH
            b = c % 2
            csl = pl.ds((c % NCH) * NC, NC)
            if c >= 2:
                cps[c - 2].wait()
            y = comm_ref[final_slot, d, :, csl].astype(jnp.float32)
            q = jnp.clip(jnp.round(y / scale), 0.0, 127.0)
            obuf[b] = (q * scale).astype(jnp.bfloat16)
            cp = pltpu.make_async_copy(
                obuf.at[b],
                out_hbm.at[:, pl.ds(d * HALF + (c % NCH) * NC, NC)],
                odma_sems.at[b])
            cp.start()
            cps.append(cp)
        cps[-2].wait()
        cps[-1].wait()

    return pl.pallas_call(
        body,
        out_shape=jax.ShapeDtypeStruct((M_BLK, N_TOT), jnp.bfloat16),
        in_specs=[
            pl.BlockSpec(memory_space=pl.ANY),
            pl.BlockSpec(memory_space=pl.ANY),
        ],
        out_specs=pl.BlockSpec(memory_space=pl.ANY),
        scratch_shapes=[
            pltpu.VMEM((2, 2, M_BLK, HALF), jnp.bfloat16),
            pltpu.VMEM((2, M_BLK, HALF), jnp.bfloat16),
            pltpu.VMEM((2, M_BLK, 1024), jnp.bfloat16),
            pltpu.VMEM((2, 1024, NC), jnp.float32),
            pltpu.VMEM((2, M_BLK, NC), jnp.bfloat16),
            pltpu.VMEM((N_DEV, 128), jnp.float32),
            pltpu.SemaphoreType.DMA((2,)),
            pltpu.SemaphoreType.DMA((2,)),
            pltpu.SemaphoreType.DMA((2,)),
            pltpu.SemaphoreType.DMA((2,)),
            pltpu.SemaphoreType.DMA((4,)),
            pltpu.SemaphoreType.DMA((4,)),
            pltpu.SemaphoreType.DMA((N_DEV - 1,)),
            pltpu.SemaphoreType.DMA((N_DEV - 1,)),
            pltpu.SemaphoreType.DMA((2,)),
            pltpu.SemaphoreType.DMA((2,)),
            pltpu.SemaphoreType.DMA((2,)),
        ],
        compiler_params=pltpu.CompilerParams(
            collective_id=0,
            vmem_limit_bytes=63 * 1024 * 1024,
        ),
    )(x, w_mat)


# device time: 354986 ns/iter; 2.2603x vs baseline; 1.0233x over previous
import jax
import jax.numpy as jnp
from jax import lax
from jax.experimental import pallas as pl
from jax.experimental.pallas import tpu as pltpu

N_DEV = 4
M_BLK = 1024
N_TOT = 8192
HALF = N_TOT // 2
NC = 256
NCH = HALF // NC
SEED_SUB = NCH // 2


def kernel(x, w_mat):

    def body(x_hbm, w_hbm, out_hbm, comm_ref, pacc_ref, xbuf, wbuf, obuf,
             amax_ref, sendR, recvR, sendL, recvL, s1send, s1recv,
             a_send_sems, a_recv_sems, xsems, wsems, odma_sems):
        p = lax.axis_index("i")
        left = lax.rem(p + N_DEV - 1, N_DEV)
        right = lax.rem(p + 1, N_DEV)

        barrier_sem = pltpu.get_barrier_semaphore()
        for nbr in (left, right):
            pl.semaphore_signal(barrier_sem, inc=1, device_id=(nbr,),
                                device_id_type=pl.DeviceIdType.MESH)
        pl.semaphore_wait(barrier_sem, 2)

        def dir_partials(j, d, dst_ref, c0=0, c1=NCH, load_x=True):
            base = d * HALF
            if load_x:
                xcp = pltpu.make_async_copy(
                    x_hbm.at[pl.ds(j * M_BLK, M_BLK), :], xbuf.at[d],
                    xsems.at[d])
                xcp.start()
            pltpu.make_async_copy(
                w_hbm.at[:, pl.ds(base + c0 * NC, NC)],
                wbuf.at[c0 % 2], wsems.at[c0 % 2]).start()
            if load_x:
                xcp.wait()
            x_bf = xbuf[d].astype(jnp.bfloat16)

            def chunk(c, _):
                buf = lax.rem(c, 2)
                nbuf = lax.rem(c + 1, 2)

                @pl.when(c + 1 < c1)
                def _():
                    pltpu.make_async_copy(
                        w_hbm.at[:, pl.ds(base + (c + 1) * NC, NC)],
                        wbuf.at[nbuf], wsems.at[nbuf]).start()

                pltpu.make_async_copy(
                    w_hbm.at[:, pl.ds(base + c * NC, NC)],
                    wbuf.at[buf], wsems.at[buf]).wait()
                acc = lax.dot_general(
                    x_bf, wbuf[buf].astype(jnp.bfloat16),
                    (((1,), (0,)), ((), ())),
                    preferred_element_type=jnp.float32)
                dst_ref[d, :, pl.ds(c * NC, NC)] = acc.astype(jnp.bfloat16)
                return 0

            lax.fori_loop(c0, c1, chunk, 0)

        def hop1_sub(d, sub, nbr):
            i = d * 2 + sub
            cols = pl.ds(sub * SEED_SUB * NC, SEED_SUB * NC)
            return pltpu.make_async_remote_copy(
                src_ref=comm_ref.at[0, d, :, cols],
                dst_ref=comm_ref.at[1, d, :, cols],
                send_sem=s1send.at[i],
                recv_sem=s1recv.at[i],
                device_id=(nbr,),
                device_id_type=pl.DeviceIdType.MESH,
            )

        def hop_full(s, d, nbr):
            send_slot = (s - 1) % 2
            recv_slot = s % 2
            ssem, rsem = (sendR, recvR) if d == 0 else (sendL, recvL)
            return pltpu.make_async_remote_copy(
                src_ref=comm_ref.at[send_slot, d],
                dst_ref=comm_ref.at[recv_slot, d],
                send_sem=ssem.at[send_slot],
                recv_sem=rsem.at[recv_slot],
                device_id=(nbr,),
                device_id_type=pl.DeviceIdType.MESH,
            )

        hop1 = {}
        dir_partials(left, 0, comm_ref.at[0], 0, SEED_SUB)
        hop1[(0, 0)] = hop1_sub(0, 0, right)
        hop1[(0, 0)].start()
        dir_partials(right, 1, comm_ref.at[0], 0, SEED_SUB)
        hop1[(1, 0)] = hop1_sub(1, 0, left)
        hop1[(1, 0)].start()
        dir_partials(left, 0, comm_ref.at[0], SEED_SUB, NCH, load_x=False)
        hop1[(0, 1)] = hop1_sub(0, 1, right)
        hop1[(0, 1)].start()
        dir_partials(right, 1, comm_ref.at[0], SEED_SUB, NCH, load_x=False)
        hop1[(1, 1)] = hop1_sub(1, 1, left)
        hop1[(1, 1)].start()

        def add_dir(d, recv_slot):
            def add_chunk(c, _):
                csl = pl.ds(c * NC, NC)
                acc = (comm_ref[recv_slot, d, :, csl].astype(jnp.float32)
                       + pacc_ref[d, :, csl].astype(jnp.float32))
                comm_ref[recv_slot, d, :, csl] = acc.astype(jnp.bfloat16)
                return 0
            lax.fori_loop(0, NCH, add_chunk, 0)

        amax = jnp.float32(0.0)
        nxt = {}
        for s in range(1, N_DEV):
            recv_slot = s % 2
            jr = lax.rem(p + 2 * N_DEV - 1 - s, N_DEV)
            jl = lax.rem(p + 1 + s, N_DEV)
            dir_partials(jr, 0, pacc_ref)
            dir_partials(jl, 1, pacc_ref)

            if s == 1:
                waitsR = [hop1[(0, 0)], hop1[(0, 1)]]
                waitsL = [hop1[(1, 0)], hop1[(1, 1)]]
            else:
                waitsR = [nxt[0]]
                waitsL = [nxt[1]]

            if s < N_DEV - 1:
                for d, waits, nbr in ((0, waitsR, right), (1, waitsL, left)):
                    for rd in waits:
                        rd.wait()
                    add_dir(d, recv_slot)
                    nxt[d] = hop_full(s + 1, d, nbr)
                    nxt[d].start()
            else:
                for rd in waitsR + waitsL:
                    rd.wait()

                def fin_chunk(c, am):
                    d = c // NCH
                    csl = pl.ds(lax.rem(c, NCH) * NC, NC)
                    y = (comm_ref[recv_slot, d, :, csl].astype(jnp.float32)
                         + pacc_ref[d, :, csl].astype(jnp.float32))
                    y = jnp.maximum(y, 0.0)
                    comm_ref[recv_slot, d, :, csl] = y.astype(jnp.bfloat16)
                    return jnp.maximum(am, jnp.max(y))
                amax = lax.fori_loop(0, 2 * NCH, fin_chunk, amax)
        final_slot = (N_DEV - 1) % 2

        amax_ref[pl.ds(p, 1), :] = jnp.full((1, 128), amax, jnp.float32)
        sends = []
        for off in range(1, N_DEV):
            tgt = lax.rem(p + off, N_DEV)
            a = pltpu.make_async_remote_copy(
                src_ref=amax_ref.at[pl.ds(p, 1)],
                dst_ref=amax_ref.at[pl.ds(p, 1)],
                send_sem=a_send_sems.at[off - 1],
                recv_sem=a_recv_sems.at[off - 1],
                device_id=(tgt,),
                device_id_type=pl.DeviceIdType.MESH,
            )
            a.start()
            sends.append(a)
        for off in range(1, N_DEV):
            src = lax.rem(p + N_DEV - off, N_DEV)
            r = pltpu.make_async_remote_copy(
                src_ref=amax_ref.at[pl.ds(p, 1)],
                dst_ref=amax_ref.at[pl.ds(src, 1)],
                send_sem=a_send_sems.at[off - 1],
                recv_sem=a_recv_sems.at[off - 1],
                device_id=(src,),
                device_id_type=pl.DeviceIdType.MESH,
            )
            r.wait_recv()
        for a in sends:
            a.wait_send()

        amax_g = jnp.max(amax_ref[:, :])
        scale = amax_g / 127.0

        cps = []
        for c in range(2 * NCH):
            d = c // NCH
            b = c % 2
            csl = pl.ds((c % NCH) * NC, NC)
            if c >= 2:
                cps[c - 2].wait()
            y = comm_ref[final_slot, d, :, csl].astype(jnp.float32)
            q = jnp.clip(jnp.round(y / scale), 0.0, 127.0)
            obuf[b] = (q * scale).astype(jnp.bfloat16)
            cp = pltpu.make_async_copy(
                obuf.at[b],
                out_hbm.at[:, pl.ds(d * HALF + (c % NCH) * NC, NC)],
                odma_sems.at[b])
            cp.start()
            cps.append(cp)
        cps[-2].wait()
        cps[-1].wait()

    return pl.pallas_call(
        body,
        out_shape=jax.ShapeDtypeStruct((M_BLK, N_TOT), jnp.bfloat16),
        in_specs=[
            pl.BlockSpec(memory_space=pl.ANY),
            pl.BlockSpec(memory_space=pl.ANY),
        ],
        out_specs=pl.BlockSpec(memory_space=pl.ANY),
        scratch_shapes=[
            pltpu.VMEM((2, 2, M_BLK, HALF), jnp.bfloat16),
            pltpu.VMEM((2, M_BLK, HALF), jnp.bfloat16),
            pltpu.VMEM((2, M_BLK, 1024), jnp.float32),
            pltpu.VMEM((2, 1024, NC), jnp.float32),
            pltpu.VMEM((2, M_BLK, NC), jnp.bfloat16),
            pltpu.VMEM((N_DEV, 128), jnp.float32),
            pltpu.SemaphoreType.DMA((2,)),
            pltpu.SemaphoreType.DMA((2,)),
            pltpu.SemaphoreType.DMA((2,)),
            pltpu.SemaphoreType.DMA((2,)),
            pltpu.SemaphoreType.DMA((4,)),
            pltpu.SemaphoreType.DMA((4,)),
            pltpu.SemaphoreType.DMA((N_DEV - 1,)),
            pltpu.SemaphoreType.DMA((N_DEV - 1,)),
            pltpu.SemaphoreType.DMA((2,)),
            pltpu.SemaphoreType.DMA((2,)),
            pltpu.SemaphoreType.DMA((2,)),
        ],
        compiler_params=pltpu.CompilerParams(
            collective_id=0,
            vmem_limit_bytes=67000000,
        ),
    )(x, w_mat)
